# Initial kernel scaffold; baseline (speedup 1.0000x reference)
import functools

import jax
import jax.numpy as jnp
from jax import lax
from jax.experimental import pallas as pl
from jax.experimental.pallas import tpu as pltpu
from jax.experimental.pallas import tpu_sc as plsc

N_NODES = 10000
N_EDGES = 320000
IN_DIM = 128
HIDDEN = 64

NC = 2
NS = 16
NW = NC * NS
EDGES_PER_TILE = N_EDGES // NW
CHUNK = 80
NCHUNKS = EDGES_PER_TILE // CHUNK
ROWS_PER_TILE = N_NODES // NS
ZROWS = 125

DEG_PAD = 10240
DEG_PER_TILE = DEG_PAD // NS

_mesh = plsc.VectorSubcoreMesh(core_axis_name="c", subcore_axis_name="s")


def _deg_body(dst_hbm, out_hbm, acc_sh, dst_v, ones_v, zero_v, sem):
    c = lax.axis_index("c")
    s = lax.axis_index("s")
    wid = c * NS + s

    def fill(i, _):
        zero_v[pl.ds(i * 16, 16)] = jnp.zeros((16,), jnp.float32)
        ones_v[pl.ds((i % 5) * 16, 16)] = jnp.ones((16,), jnp.float32)
        return 0

    lax.fori_loop(0, DEG_PER_TILE // 16, fill, 0)
    pltpu.sync_copy(zero_v, acc_sh.at[pl.ds(s * DEG_PER_TILE, DEG_PER_TILE)])
    plsc.subcore_barrier()

    base = wid * EDGES_PER_TILE

    def chunk(j, _):
        off = pl.multiple_of(base + j * CHUNK, 8)
        pltpu.sync_copy(dst_hbm.at[pl.ds(off, CHUNK)], dst_v)
        pltpu.sync_copy(ones_v, acc_sh.at[dst_v], add=True)
        return 0

    lax.fori_loop(0, NCHUNKS, chunk, 0)
    plsc.subcore_barrier()
    sl = pl.ds(s * DEG_PER_TILE, DEG_PER_TILE)
    pltpu.sync_copy(acc_sh.at[sl], out_hbm.at[c, sl])


_deg_call = pl.kernel(
    _deg_body,
    out_type=jax.ShapeDtypeStruct((NC, DEG_PAD), jnp.float32),
    mesh=_mesh,
    scratch_types=[
        pltpu.VMEM_SHARED((DEG_PAD,), jnp.float32),
        pltpu.VMEM((CHUNK,), jnp.int32),
        pltpu.VMEM((CHUNK,), jnp.float32),
        pltpu.VMEM((DEG_PER_TILE,), jnp.float32),
        pltpu.SemaphoreType.DMA,
    ],
)


def _agg_body(h_hbm, src_hbm, dst_hbm, out_hbm,
              acc_sh, src_v, dst_v, rows_v, zero_v, sem):
    c = lax.axis_index("c")
    s = lax.axis_index("s")
    wid = c * NS + s

    def fill(t, _):
        zero_v[t // 4, pl.ds((t % 4) * 16, 16)] = jnp.zeros((16,), jnp.float32)
        return 0

    lax.fori_loop(0, ZROWS * 4, fill, 0)
    for q in range(ROWS_PER_TILE // ZROWS):
        pltpu.sync_copy(
            zero_v, acc_sh.at[pl.ds(s * ROWS_PER_TILE + q * ZROWS, ZROWS)])
    plsc.subcore_barrier()

    base = wid * EDGES_PER_TILE

    def chunk(j, _):
        off = pl.multiple_of(base + j * CHUNK, 8)
        pltpu.sync_copy(src_hbm.at[pl.ds(off, CHUNK)], src_v)
        pltpu.sync_copy(dst_hbm.at[pl.ds(off, CHUNK)], dst_v)
        pltpu.async_copy(h_hbm.at[src_v], rows_v, sem).wait()
        pltpu.sync_copy(rows_v, acc_sh.at[dst_v], add=True)
        return 0

    lax.fori_loop(0, NCHUNKS, chunk, 0)
    plsc.subcore_barrier()
    sl = pl.ds(s * ROWS_PER_TILE, ROWS_PER_TILE)
    pltpu.sync_copy(acc_sh.at[sl], out_hbm.at[c, sl])


_agg_call = pl.kernel(
    _agg_body,
    out_type=jax.ShapeDtypeStruct((NC, N_NODES, HIDDEN), jnp.float32),
    mesh=_mesh,
    scratch_types=[
        pltpu.VMEM_SHARED((N_NODES, HIDDEN), jnp.float32),
        pltpu.VMEM((CHUNK,), jnp.int32),
        pltpu.VMEM((CHUNK,), jnp.int32),
        pltpu.VMEM((CHUNK, HIDDEN), jnp.float32),
        pltpu.VMEM((ZROWS, HIDDEN), jnp.float32),
        pltpu.SemaphoreType.DMA,
    ],
)


ROW_BLK = 1000
_GRID = N_NODES // ROW_BLK


def _mm_scale_body(x_ref, w_ref, d_ref, h1_ref, h1s_ref):
    h1 = jnp.dot(x_ref[...], w_ref[...], preferred_element_type=jnp.float32)
    h1_ref[...] = h1
    h1s_ref[...] = h1 * d_ref[...]


def _mm_scale(x, W1, dinv2d):
    return pl.pallas_call(
        _mm_scale_body,
        grid=(_GRID,),
        in_specs=[
            pl.BlockSpec((ROW_BLK, IN_DIM), lambda i: (i, 0)),
            pl.BlockSpec((IN_DIM, HIDDEN), lambda i: (0, 0)),
            pl.BlockSpec((ROW_BLK, 1), lambda i: (i, 0)),
        ],
        out_specs=[
            pl.BlockSpec((ROW_BLK, HIDDEN), lambda i: (i, 0)),
            pl.BlockSpec((ROW_BLK, HIDDEN), lambda i: (i, 0)),
        ],
        out_shape=[jax.ShapeDtypeStruct((N_NODES, HIDDEN), jnp.float32)] * 2,
    )(x, W1, dinv2d)


def _layer1_post_body(acc_ref, h1_ref, d_ref, b_ref, H_ref, Hs_ref):
    d = d_ref[...]
    agg = acc_ref[0] + acc_ref[1]
    pre = d * agg + (d * d) * h1_ref[...] + b_ref[...]
    H = jnp.maximum(pre, 0.0)
    H_ref[...] = H
    Hs_ref[...] = H * d


def _layer1_post(accp, h1, dinv2d, b1row):
    return pl.pallas_call(
        _layer1_post_body,
        grid=(_GRID,),
        in_specs=[
            pl.BlockSpec((NC, ROW_BLK, HIDDEN), lambda i: (0, i, 0)),
            pl.BlockSpec((ROW_BLK, HIDDEN), lambda i: (i, 0)),
            pl.BlockSpec((ROW_BLK, 1), lambda i: (i, 0)),
            pl.BlockSpec((1, HIDDEN), lambda i: (0, 0)),
        ],
        out_specs=[
            pl.BlockSpec((ROW_BLK, HIDDEN), lambda i: (i, 0)),
            pl.BlockSpec((ROW_BLK, HIDDEN), lambda i: (i, 0)),
        ],
        out_shape=[jax.ShapeDtypeStruct((N_NODES, HIDDEN), jnp.float32)] * 2,
    )(accp, h1, dinv2d, b1row)


OUT_PAD = 128


def _layer2_post_body(acc_ref, H_ref, d_ref, w_ref, b_ref, out_ref):
    d = d_ref[...]
    tmp = d * (acc_ref[0] + acc_ref[1]) + (d * d) * H_ref[...]
    out_ref[...] = (
        jnp.dot(tmp, w_ref[...], preferred_element_type=jnp.float32)
        + b_ref[...])


def _layer2_post(accp, H, dinv2d, W2p, b2row):
    return pl.pallas_call(
        _layer2_post_body,
        grid=(_GRID,),
        in_specs=[
            pl.BlockSpec((NC, ROW_BLK, HIDDEN), lambda i: (0, i, 0)),
            pl.BlockSpec((ROW_BLK, HIDDEN), lambda i: (i, 0)),
            pl.BlockSpec((ROW_BLK, 1), lambda i: (i, 0)),
            pl.BlockSpec((HIDDEN, OUT_PAD), lambda i: (0, 0)),
            pl.BlockSpec((1, OUT_PAD), lambda i: (0, 0)),
        ],
        out_specs=pl.BlockSpec((ROW_BLK, OUT_PAD), lambda i: (i, 0)),
        out_shape=jax.ShapeDtypeStruct((N_NODES, OUT_PAD), jnp.float32),
    )(accp, H, dinv2d, W2p, b2p := None) if False else pl.pallas_call(
        _layer2_post_body,
        grid=(_GRID,),
        in_specs=[
            pl.BlockSpec((NC, ROW_BLK, HIDDEN), lambda i: (0, i, 0)),
            pl.BlockSpec((ROW_BLK, HIDDEN), lambda i: (i, 0)),
            pl.BlockSpec((ROW_BLK, 1), lambda i: (i, 0)),
            pl.BlockSpec((HIDDEN, OUT_PAD), lambda i: (0, 0)),
            pl.BlockSpec((1, OUT_PAD), lambda i: (0, 0)),
        ],
        out_specs=pl.BlockSpec((ROW_BLK, OUT_PAD), lambda i: (i, 0)),
        out_shape=jax.ShapeDtypeStruct((N_NODES, OUT_PAD), jnp.float32),
    )(accp, H, dinv2d, W2p, b2row)


@jax.jit
def kernel(x, edge_index, W1, b1, W2, b2):
    ei = edge_index.astype(jnp.int32)
    src = ei[0]
    dst = ei[1]

    degp = _deg_call(dst)
    deg = degp[0, :N_NODES] + degp[1, :N_NODES] + 1.0
    dinv2d = lax.rsqrt(deg)[:, None]

    h1, h1s = _mm_scale(x, W1, dinv2d)
    accp1 = _agg_call(h1s, src, dst)
    H, Hs = _layer1_post(accp1, h1, dinv2d, b1[None, :])
    accp2 = _agg_call(Hs, src, dst)

    nclass = W2.shape[1]
    W2p = jnp.zeros((HIDDEN, OUT_PAD), jnp.float32).at[:, :nclass].set(W2)
    b2p = jnp.zeros((1, OUT_PAD), jnp.float32).at[0, :nclass].set(b2)
    outp = _layer2_post(accp2, H, dinv2d, W2p, b2p)
    return outp[:, :nclass]


# speedup vs baseline: 14.3594x; 14.3594x over previous
import functools

import jax
import jax.numpy as jnp
from jax import lax
from jax.experimental import pallas as pl
from jax.experimental.pallas import tpu as pltpu
from jax.experimental.pallas import tpu_sc as plsc

N_NODES = 10000
N_EDGES = 320000
IN_DIM = 128
HIDDEN = 64

NC = 2
NS = 16
NW = NC * NS
EDGES_PER_TILE = N_EDGES // NW
CHUNK = 80
NCHUNKS = EDGES_PER_TILE // CHUNK
ACC_PAD = 10240
ROWS_PER_TILE = ACC_PAD // NS
ZROWS = 128

DEG_PAD = 10240
DEG_PER_TILE = DEG_PAD // NS

_mesh = plsc.VectorSubcoreMesh(core_axis_name="c", subcore_axis_name="s")


def _deg_body(dst_hbm, out_hbm, acc_sh, dst_v, ones_v, zero_v, sem):
    c = lax.axis_index("c")
    s = lax.axis_index("s")
    wid = c * NS + s

    def fill(i, _):
        zero_v[pl.ds(i * 16, 16)] = jnp.zeros((16,), jnp.float32)
        ones_v[pl.ds((i % 5) * 16, 16)] = jnp.ones((16,), jnp.float32)
        return 0

    lax.fori_loop(0, DEG_PER_TILE // 16, fill, 0)
    pltpu.sync_copy(zero_v, acc_sh.at[pl.ds(s * DEG_PER_TILE, DEG_PER_TILE)])
    plsc.subcore_barrier()

    base = wid * EDGES_PER_TILE

    def chunk(j, _):
        off = pl.multiple_of(base + j * CHUNK, 8)
        pltpu.sync_copy(dst_hbm.at[pl.ds(off, CHUNK)], dst_v)
        pltpu.sync_copy(ones_v, acc_sh.at[dst_v], add=True)
        return 0

    lax.fori_loop(0, NCHUNKS, chunk, 0)
    plsc.subcore_barrier()
    sl = pl.ds(s * DEG_PER_TILE, DEG_PER_TILE)
    pltpu.sync_copy(acc_sh.at[sl], out_hbm.at[c, sl])


_deg_call = pl.kernel(
    _deg_body,
    out_type=jax.ShapeDtypeStruct((NC, DEG_PAD), jnp.float32),
    mesh=_mesh,
    compiler_params=pltpu.CompilerParams(use_tc_tiling_on_sc=False),
    scratch_types=[
        pltpu.VMEM_SHARED((DEG_PAD,), jnp.float32),
        pltpu.VMEM((CHUNK,), jnp.int32),
        pltpu.VMEM((CHUNK,), jnp.float32),
        pltpu.VMEM((DEG_PER_TILE,), jnp.float32),
        pltpu.SemaphoreType.DMA,
    ],
)


def _agg_body(h_hbm, src_hbm, dst_hbm, out_hbm,
              acc_sh, src_v, dst_v, rows_v, zero_v, sem):
    c = lax.axis_index("c")
    s = lax.axis_index("s")
    wid = c * NS + s

    def fill(t, _):
        zero_v[t // 4, pl.ds((t % 4) * 16, 16)] = jnp.zeros((16,), jnp.float32)
        return 0

    lax.fori_loop(0, ZROWS * 4, fill, 0)
    for q in range(ROWS_PER_TILE // ZROWS):
        pltpu.sync_copy(
            zero_v, acc_sh.at[pl.ds(s * ROWS_PER_TILE + q * ZROWS, ZROWS)])
    plsc.subcore_barrier()

    base = wid * EDGES_PER_TILE

    def chunk(j, _):
        off = pl.multiple_of(base + j * CHUNK, 8)
        pltpu.sync_copy(src_hbm.at[pl.ds(off, CHUNK)], src_v)
        pltpu.sync_copy(dst_hbm.at[pl.ds(off, CHUNK)], dst_v)
        pltpu.async_copy(h_hbm.at[src_v], rows_v, sem).wait()
        pltpu.sync_copy(rows_v, acc_sh.at[dst_v], add=True)
        return 0

    lax.fori_loop(0, NCHUNKS, chunk, 0)
    plsc.subcore_barrier()
    sl = pl.ds(s * ROWS_PER_TILE, ROWS_PER_TILE)
    pltpu.sync_copy(acc_sh.at[sl], out_hbm.at[c, sl])


_agg_call = pl.kernel(
    _agg_body,
    out_type=jax.ShapeDtypeStruct((NC, ACC_PAD, HIDDEN), jnp.float32),
    mesh=_mesh,
    compiler_params=pltpu.CompilerParams(use_tc_tiling_on_sc=False),
    scratch_types=[
        pltpu.VMEM_SHARED((ACC_PAD, HIDDEN), jnp.float32),
        pltpu.VMEM((CHUNK,), jnp.int32),
        pltpu.VMEM((CHUNK,), jnp.int32),
        pltpu.VMEM((CHUNK, HIDDEN), jnp.float32),
        pltpu.VMEM((ZROWS, HIDDEN), jnp.float32),
        pltpu.SemaphoreType.DMA,
    ],
)


ROW_BLK = 1000
_GRID = N_NODES // ROW_BLK


def _mm_scale_body(x_ref, w_ref, d_ref, h1_ref, h1s_ref):
    h1 = jnp.dot(x_ref[...], w_ref[...], preferred_element_type=jnp.float32)
    h1_ref[...] = h1
    h1s_ref[...] = h1 * d_ref[...]


def _mm_scale(x, W1, dinv2d):
    return pl.pallas_call(
        _mm_scale_body,
        grid=(_GRID,),
        in_specs=[
            pl.BlockSpec((ROW_BLK, IN_DIM), lambda i: (i, 0)),
            pl.BlockSpec((IN_DIM, HIDDEN), lambda i: (0, 0)),
            pl.BlockSpec((ROW_BLK, 1), lambda i: (i, 0)),
        ],
        out_specs=[
            pl.BlockSpec((ROW_BLK, HIDDEN), lambda i: (i, 0)),
            pl.BlockSpec((ROW_BLK, HIDDEN), lambda i: (i, 0)),
        ],
        out_shape=[jax.ShapeDtypeStruct((N_NODES, HIDDEN), jnp.float32)] * 2,
    )(x, W1, dinv2d)


def _layer1_post_body(acc_ref, h1_ref, d_ref, b_ref, H_ref, Hs_ref):
    d = d_ref[...]
    agg = acc_ref[0] + acc_ref[1]
    pre = d * agg + (d * d) * h1_ref[...] + b_ref[...]
    H = jnp.maximum(pre, 0.0)
    H_ref[...] = H
    Hs_ref[...] = H * d


def _layer1_post(accp, h1, dinv2d, b1row):
    return pl.pallas_call(
        _layer1_post_body,
        grid=(_GRID,),
        in_specs=[
            pl.BlockSpec((NC, ROW_BLK, HIDDEN), lambda i: (0, i, 0)),
            pl.BlockSpec((ROW_BLK, HIDDEN), lambda i: (i, 0)),
            pl.BlockSpec((ROW_BLK, 1), lambda i: (i, 0)),
            pl.BlockSpec((1, HIDDEN), lambda i: (0, 0)),
        ],
        out_specs=[
            pl.BlockSpec((ROW_BLK, HIDDEN), lambda i: (i, 0)),
            pl.BlockSpec((ROW_BLK, HIDDEN), lambda i: (i, 0)),
        ],
        out_shape=[jax.ShapeDtypeStruct((N_NODES, HIDDEN), jnp.float32)] * 2,
    )(accp, h1, dinv2d, b1row)


OUT_PAD = 128


def _layer2_post_body(acc_ref, H_ref, d_ref, w_ref, b_ref, out_ref):
    d = d_ref[...]
    tmp = d * (acc_ref[0] + acc_ref[1]) + (d * d) * H_ref[...]
    out_ref[...] = (
        jnp.dot(tmp, w_ref[...], preferred_element_type=jnp.float32)
        + b_ref[...])


def _layer2_post(accp, H, dinv2d, W2p, b2row):
    return pl.pallas_call(
        _layer2_post_body,
        grid=(_GRID,),
        in_specs=[
            pl.BlockSpec((NC, ROW_BLK, HIDDEN), lambda i: (0, i, 0)),
            pl.BlockSpec((ROW_BLK, HIDDEN), lambda i: (i, 0)),
            pl.BlockSpec((ROW_BLK, 1), lambda i: (i, 0)),
            pl.BlockSpec((HIDDEN, OUT_PAD), lambda i: (0, 0)),
            pl.BlockSpec((1, OUT_PAD), lambda i: (0, 0)),
        ],
        out_specs=pl.BlockSpec((ROW_BLK, OUT_PAD), lambda i: (i, 0)),
        out_shape=jax.ShapeDtypeStruct((N_NODES, OUT_PAD), jnp.float32),
    )(accp, H, dinv2d, W2p, b2row)


@jax.jit
def kernel(x, edge_index, W1, b1, W2, b2):
    ei = edge_index.astype(jnp.int32)
    src = ei[0]
    dst = ei[1]

    degp = _deg_call(dst)
    deg = degp[0, :N_NODES] + degp[1, :N_NODES] + 1.0
    dinv2d = lax.rsqrt(deg)[:, None]

    h1, h1s = _mm_scale(x, W1, dinv2d)
    accp1 = _agg_call(h1s, src, dst)[:, :N_NODES]
    H, Hs = _layer1_post(accp1, h1, dinv2d, b1[None, :])
    accp2 = _agg_call(Hs, src, dst)[:, :N_NODES]

    nclass = W2.shape[1]
    W2p = jnp.zeros((HIDDEN, OUT_PAD), jnp.float32).at[:, :nclass].set(W2)
    b2p = jnp.zeros((1, OUT_PAD), jnp.float32).at[0, :nclass].set(b2)
    outp = _layer2_post(accp2, H, dinv2d, W2p, b2p)
    return outp[:, :nclass]
